# Initial kernel scaffold; baseline (speedup 1.0000x reference)
import functools

import jax
import jax.numpy as jnp
from jax import lax
from jax.experimental import pallas as pl
from jax.experimental.pallas import tpu as pltpu
from jax.experimental.pallas import tpu_sc as plsc

NC = 2
NS = 16
NW = NC * NS
L = 16
CH = 128


def _sc_aggregate(ego_embed, head, tail, etype, relation_weight,
                  n_acc, chunks_per_worker):
  n_nodes, d = ego_embed.shape
  rpt = n_acc // NS
  wb = rpt // CH
  mesh = plsc.VectorSubcoreMesh(core_axis_name="c", subcore_axis_name="s",
                                num_cores=NC, num_subcores=NS)

  @functools.partial(
      pl.kernel,
      out_type=(
          jax.ShapeDtypeStruct((NC * n_acc, d), jnp.float32),
          jax.ShapeDtypeStruct((NC * n_acc, L), jnp.float32),
      ),
      mesh=mesh,
      scratch_types=[
          pltpu.VMEM((CH,), jnp.int32),
          pltpu.VMEM((CH,), jnp.int32),
          pltpu.VMEM((CH,), jnp.int32),
          pltpu.VMEM((CH, d), jnp.float32),
          pltpu.VMEM((CH, d), jnp.float32),
          pltpu.VMEM((CH, L), jnp.float32),
          pltpu.VMEM_SHARED((n_acc, d), jnp.float32),
          pltpu.VMEM_SHARED((n_acc, L), jnp.float32),
          pltpu.SemaphoreType.DMA,
      ],
  )
  def agg(ego_hbm, head_hbm, tail_hbm, etype_hbm, relw_hbm,
          out_val, out_cnt, tail_idx, head_idx, type_idx,
          tail_rows, rel_rows, cnt_rows, acc_val, acc_cnt, sem):
    c = lax.axis_index("c")
    s = lax.axis_index("s")
    wid = c * NS + s
    zv = jnp.zeros((L,), jnp.float32)

    def zero_body(i, _):
      for j in range(d // L):
        tail_rows[i, pl.ds(j * L, L)] = zv
      cnt_rows[i, :] = zv
      return 0
    lax.fori_loop(0, CH, zero_body, 0)
    row0 = s * rpt
    for k in range(wb):
      pltpu.sync_copy(tail_rows, acc_val.at[pl.ds(row0 + k * CH, CH)])
      pltpu.sync_copy(cnt_rows, acc_cnt.at[pl.ds(row0 + k * CH, CH)])

    one_hot = jnp.where(lax.iota(jnp.int32, L) == 0,
                        jnp.float32(1.0), jnp.float32(0.0))
    def oh_body(i, _):
      cnt_rows[i, :] = one_hot
      return 0
    lax.fori_loop(0, CH, oh_body, 0)

    plsc.subcore_barrier()

    e_base = wid * (chunks_per_worker * CH)

    def chunk_body(g, _):
      base = e_base + g * CH
      pltpu.sync_copy(tail_hbm.at[pl.ds(base, CH)], tail_idx)
      pltpu.sync_copy(etype_hbm.at[pl.ds(base, CH)], type_idx)
      pltpu.sync_copy(head_hbm.at[pl.ds(base, CH)], head_idx)
      pltpu.async_copy(ego_hbm.at[tail_idx], tail_rows, sem).wait()
      pltpu.async_copy(relw_hbm.at[type_idx], rel_rows, sem).wait()

      def mul_body(i, _):
        for j in range(d // L):
          sl = pl.ds(j * L, L)
          rel_rows[i, sl] = rel_rows[i, sl] * tail_rows[i, sl]
        return 0
      lax.fori_loop(0, CH, mul_body, 0)

      pltpu.sync_copy(rel_rows, acc_val.at[head_idx], add=True)
      pltpu.sync_copy(cnt_rows, acc_cnt.at[head_idx], add=True)
      return 0
    lax.fori_loop(0, chunks_per_worker, chunk_body, 0)

    plsc.subcore_barrier()

    out_base = c * n_acc + row0
    for k in range(wb):
      pltpu.sync_copy(acc_val.at[pl.ds(row0 + k * CH, CH)], tail_rows)
      pltpu.sync_copy(tail_rows, out_val.at[pl.ds(out_base + k * CH, CH)])
      pltpu.sync_copy(acc_cnt.at[pl.ds(row0 + k * CH, CH)], cnt_rows)
      pltpu.sync_copy(cnt_rows, out_cnt.at[pl.ds(out_base + k * CH, CH)])

  return agg(ego_embed, head, tail, etype, relation_weight)


def _combine_body(v_ref, c_ref, o_ref):
  v = v_ref[...]
  cnt = c_ref[...]
  deg = cnt[0, :, :1] + cnt[1, :, :1]
  o_ref[...] = (v[0] + v[1]) / jnp.maximum(deg, 1.0)


def kernel(ego_embed, edge_index, edge_type, relation_weight):
  n_nodes, d = ego_embed.shape
  e = edge_index.shape[1]
  head = edge_index[0].astype(jnp.int32)
  tail = edge_index[1].astype(jnp.int32)
  etype = edge_type.astype(jnp.int32)

  quantum = NW * CH
  e_pad = ((e + quantum - 1) // quantum) * quantum
  if e_pad > e:
    pad = e_pad - e
    head = jnp.concatenate([head, jnp.full((pad,), n_nodes, jnp.int32)])
    tail = jnp.concatenate([tail, jnp.zeros((pad,), jnp.int32)])
    etype = jnp.concatenate([etype, jnp.zeros((pad,), jnp.int32)])
  chunks_per_worker = e_pad // quantum

  acc_quantum = NS * CH
  n_acc = ((n_nodes + 1 + acc_quantum - 1) // acc_quantum) * acc_quantum

  val_parts, cnt_parts = _sc_aggregate(
      ego_embed, head, tail, etype, relation_weight, n_acc,
      chunks_per_worker)
  val_parts = val_parts.reshape(NC, n_acc, d)
  cnt_parts = cnt_parts.reshape(NC, n_acc, L)

  br = 1000
  assert n_nodes % br == 0
  return pl.pallas_call(
      _combine_body,
      grid=(n_nodes // br,),
      in_specs=[
          pl.BlockSpec((NC, br, d), lambda i: (0, i, 0)),
          pl.BlockSpec((NC, br, L), lambda i: (0, i, 0)),
      ],
      out_specs=pl.BlockSpec((br, d), lambda i: (i, 0)),
      out_shape=jax.ShapeDtypeStruct((n_nodes, d), jnp.float32),
  )(val_parts, cnt_parts)


# speedup vs baseline: 3.5535x; 3.5535x over previous
import functools

import jax
import jax.numpy as jnp
from jax import lax
from jax.experimental import pallas as pl
from jax.experimental.pallas import tpu as pltpu
from jax.experimental.pallas import tpu_sc as plsc

NC = 2
NS = 16
L = 16
CH = 128


def _sc_aggregate(ego_halves, head, tail, etype, relw_halves,
                  n_nodes, n_rel, n_acc, chunks_per_tile):
  dh = ego_halves.shape[1]
  rpt = n_acc // NS
  wb_sizes = [CH] * (rpt // CH) + ([rpt % CH] if rpt % CH else [])
  mesh = plsc.VectorSubcoreMesh(core_axis_name="c", subcore_axis_name="s",
                                num_cores=NC, num_subcores=NS)

  @functools.partial(
      pl.kernel,
      out_type=(
          jax.ShapeDtypeStruct((NC * n_acc, dh), jnp.float32),
          jax.ShapeDtypeStruct((n_acc, L), jnp.float32),
      ),
      mesh=mesh,
      compiler_params=pltpu.CompilerParams(use_tc_tiling_on_sc=False),
      scratch_types=[
          pltpu.VMEM((CH,), jnp.int32),
          pltpu.VMEM((CH,), jnp.int32),
          pltpu.VMEM((CH,), jnp.int32),
          pltpu.VMEM((CH, dh), jnp.float32),
          pltpu.VMEM((CH, dh), jnp.float32),
          pltpu.VMEM((CH, L), jnp.float32),
          pltpu.VMEM_SHARED((n_acc, dh), jnp.float32),
          pltpu.VMEM_SHARED((n_acc, L), jnp.float32),
          pltpu.SemaphoreType.DMA,
      ],
  )
  def agg(ego_hbm, head_hbm, tail_hbm, etype_hbm, relw_hbm,
          out_val, out_cnt, tail_idx, head_idx, type_idx,
          tail_rows, rel_rows, cnt_rows, acc_val, acc_cnt, sem):
    c = lax.axis_index("c")
    s = lax.axis_index("s")
    zv = jnp.zeros((L,), jnp.float32)

    def zero_body(i, _):
      for j in range(dh // L):
        tail_rows[i, pl.ds(j * L, L)] = zv
      cnt_rows[i, :] = zv
      return 0
    lax.fori_loop(0, CH, zero_body, 0)
    row0 = s * rpt
    off = 0
    for sz in wb_sizes:
      pltpu.sync_copy(tail_rows.at[pl.ds(0, sz)],
                      acc_val.at[pl.ds(row0 + off, sz)])
      pltpu.sync_copy(cnt_rows.at[pl.ds(0, sz)],
                      acc_cnt.at[pl.ds(row0 + off, sz)])
      off += sz

    one_hot = jnp.where(lax.iota(jnp.int32, L) == 0,
                        jnp.float32(1.0), jnp.float32(0.0))
    def oh_body(i, _):
      cnt_rows[i, :] = one_hot
      return 0
    lax.fori_loop(0, CH, oh_body, 0)

    plsc.subcore_barrier()

    e_base = s * (chunks_per_tile * CH)
    tail_off = (c * n_nodes).astype(jnp.int32)
    type_off = (c * n_rel).astype(jnp.int32)

    def chunk_body(g, _):
      base = e_base + g * CH
      pltpu.sync_copy(tail_hbm.at[pl.ds(base, CH)], tail_idx)
      pltpu.sync_copy(etype_hbm.at[pl.ds(base, CH)], type_idx)
      pltpu.sync_copy(head_hbm.at[pl.ds(base, CH)], head_idx)
      def adj_body(k, _):
        sl = pl.ds(k * L, L)
        tail_idx[sl] = tail_idx[sl] + tail_off
        type_idx[sl] = type_idx[sl] + type_off
        return 0
      lax.fori_loop(0, CH // L, adj_body, 0)
      pltpu.async_copy(ego_hbm.at[tail_idx], tail_rows, sem).wait()
      pltpu.async_copy(relw_hbm.at[type_idx], rel_rows, sem).wait()

      def mul_body(i, _):
        for j in range(dh // L):
          sl = pl.ds(j * L, L)
          rel_rows[i, sl] = rel_rows[i, sl] * tail_rows[i, sl]
        return 0
      lax.fori_loop(0, CH, mul_body, 0)

      pltpu.sync_copy(rel_rows, acc_val.at[head_idx], add=True)
      @pl.when(c == 0)
      def _():
        pltpu.sync_copy(cnt_rows, acc_cnt.at[head_idx], add=True)
      return 0
    lax.fori_loop(0, chunks_per_tile, chunk_body, 0)

    plsc.subcore_barrier()

    out_base = c * n_acc + row0
    off = 0
    for sz in wb_sizes:
      pltpu.sync_copy(acc_val.at[pl.ds(row0 + off, sz)],
                      tail_rows.at[pl.ds(0, sz)])
      pltpu.sync_copy(tail_rows.at[pl.ds(0, sz)],
                      out_val.at[pl.ds(out_base + off, sz)])
      off += sz
    @pl.when(c == 0)
    def _():
      o = 0
      for sz in wb_sizes:
        pltpu.sync_copy(acc_cnt.at[pl.ds(row0 + o, sz)],
                        cnt_rows.at[pl.ds(0, sz)])
        pltpu.sync_copy(cnt_rows.at[pl.ds(0, sz)],
                        out_cnt.at[pl.ds(row0 + o, sz)])
        o += sz

  return agg(ego_halves, head, tail, etype, relw_halves)


def _combine_body(v_ref, c_ref, o_ref):
  v = v_ref[...]
  deg = jnp.maximum(c_ref[0, :, :1], 1.0)
  o_ref[...] = jnp.concatenate([v[0], v[1]], axis=1) / deg


def kernel(ego_embed, edge_index, edge_type, relation_weight):
  n_nodes, d = ego_embed.shape
  n_rel = relation_weight.shape[0]
  e = edge_index.shape[1]
  dh = d // NC
  head = edge_index[0].astype(jnp.int32)
  tail = edge_index[1].astype(jnp.int32)
  etype = edge_type.astype(jnp.int32)

  ego_halves = jnp.concatenate([ego_embed[:, :dh], ego_embed[:, dh:]], 0)
  relw_halves = jnp.concatenate(
      [relation_weight[:, :dh], relation_weight[:, dh:]], 0)

  quantum = NS * CH
  e_pad = ((e + quantum - 1) // quantum) * quantum
  if e_pad > e:
    pad = e_pad - e
    head = jnp.concatenate([head, jnp.full((pad,), n_nodes, jnp.int32)])
    tail = jnp.concatenate([tail, jnp.zeros((pad,), jnp.int32)])
    etype = jnp.concatenate([etype, jnp.zeros((pad,), jnp.int32)])
  chunks_per_tile = e_pad // quantum

  n_acc = ((n_nodes + 1 + NS * 8 - 1) // (NS * 8)) * (NS * 8)

  val_parts, cnt_part = _sc_aggregate(
      ego_halves, head, tail, etype, relw_halves,
      n_nodes, n_rel, n_acc, chunks_per_tile)
  val_parts = val_parts.reshape(NC, n_acc, dh)
  cnt_part = cnt_part.reshape(1, n_acc, L)

  br = 1000
  assert n_nodes % br == 0
  return pl.pallas_call(
      _combine_body,
      grid=(n_nodes // br,),
      in_specs=[
          pl.BlockSpec((NC, br, dh), lambda i: (0, i, 0)),
          pl.BlockSpec((1, br, L), lambda i: (0, i, 0)),
      ],
      out_specs=pl.BlockSpec((br, d), lambda i: (i, 0)),
      out_shape=jax.ShapeDtypeStruct((n_nodes, d), jnp.float32),
  )(val_parts, cnt_part)


# speedup vs baseline: 4.3067x; 1.2120x over previous
import functools

import jax
import jax.numpy as jnp
from jax import lax
from jax.experimental import pallas as pl
from jax.experimental.pallas import tpu as pltpu
from jax.experimental.pallas import tpu_sc as plsc

NC = 2
NS = 16
L = 16
CH = 128


def _sc_aggregate(ego_halves, head, tail, etype, relw_halves,
                  n_nodes, n_rel, n_acc, chunks_per_tile):
  dh = ego_halves.shape[1]
  rpt = n_acc // NS
  wb_sizes = [CH] * (rpt // CH) + ([rpt % CH] if rpt % CH else [])
  mesh = plsc.VectorSubcoreMesh(core_axis_name="c", subcore_axis_name="s",
                                num_cores=NC, num_subcores=NS)

  @functools.partial(
      pl.kernel,
      out_type=(
          jax.ShapeDtypeStruct((NC * n_acc, dh), jnp.float32),
          jax.ShapeDtypeStruct((n_acc, L), jnp.float32),
      ),
      mesh=mesh,
      compiler_params=pltpu.CompilerParams(use_tc_tiling_on_sc=False),
      scratch_types=[
          pltpu.VMEM((CH,), jnp.int32),
          pltpu.VMEM((CH,), jnp.int32),
          pltpu.VMEM((CH,), jnp.int32),
          pltpu.VMEM((CH, dh), jnp.float32),
          pltpu.VMEM((CH, dh), jnp.float32),
          pltpu.VMEM((CH, L), jnp.float32),
          pltpu.VMEM_SHARED((n_rel, dh), jnp.float32),
          pltpu.VMEM_SHARED((n_acc, dh), jnp.float32),
          pltpu.VMEM_SHARED((n_acc, L), jnp.float32),
          pltpu.SemaphoreType.DMA,
      ],
  )
  def agg(ego_hbm, head_hbm, tail_hbm, etype_hbm, relw_hbm,
          out_val, out_cnt, tail_idx, head_idx, type_idx,
          tail_rows, rel_rows, cnt_rows, rel_tab, acc_val, acc_cnt, sem):
    c = lax.axis_index("c")
    s = lax.axis_index("s")
    zv = jnp.zeros((L,), jnp.float32)

    def zero_body(i, _):
      for j in range(dh // L):
        tail_rows[i, pl.ds(j * L, L)] = zv
      cnt_rows[i, :] = zv
      return 0
    lax.fori_loop(0, CH, zero_body, 0)
    row0 = s * rpt
    off = 0
    for sz in wb_sizes:
      pltpu.sync_copy(tail_rows.at[pl.ds(0, sz)],
                      acc_val.at[pl.ds(row0 + off, sz)])
      pltpu.sync_copy(cnt_rows.at[pl.ds(0, sz)],
                      acc_cnt.at[pl.ds(row0 + off, sz)])
      off += sz

    one_hot = jnp.where(lax.iota(jnp.int32, L) == 0,
                        jnp.float32(1.0), jnp.float32(0.0))
    def oh_body(i, _):
      cnt_rows[i, :] = one_hot
      return 0
    lax.fori_loop(0, CH, oh_body, 0)

    @pl.when(s == 0)
    def _():
      pltpu.sync_copy(relw_hbm.at[pl.ds(c * n_rel, n_rel)], rel_tab)

    plsc.subcore_barrier()

    e_base = s * (chunks_per_tile * CH)
    tail_off = (c * n_nodes).astype(jnp.int32)

    def chunk_body(g, _):
      base = e_base + g * CH
      pltpu.sync_copy(tail_hbm.at[pl.ds(base, CH)], tail_idx)
      pltpu.sync_copy(etype_hbm.at[pl.ds(base, CH)], type_idx)
      pltpu.sync_copy(head_hbm.at[pl.ds(base, CH)], head_idx)
      def adj_body(k, _):
        sl = pl.ds(k * L, L)
        tail_idx[sl] = tail_idx[sl] + tail_off
        return 0
      lax.fori_loop(0, CH // L, adj_body, 0)
      pltpu.async_copy(ego_hbm.at[tail_idx], tail_rows, sem).wait()
      pltpu.async_copy(rel_tab.at[type_idx], rel_rows, sem).wait()

      def mul_body(i, _):
        for j in range(dh // L):
          sl = pl.ds(j * L, L)
          rel_rows[i, sl] = rel_rows[i, sl] * tail_rows[i, sl]
        return 0
      lax.fori_loop(0, CH, mul_body, 0)

      pltpu.sync_copy(rel_rows, acc_val.at[head_idx], add=True)
      @pl.when(c == 0)
      def _():
        pltpu.sync_copy(cnt_rows, acc_cnt.at[head_idx], add=True)
      return 0
    lax.fori_loop(0, chunks_per_tile, chunk_body, 0)

    plsc.subcore_barrier()

    out_base = c * n_acc + row0
    off = 0
    for sz in wb_sizes:
      pltpu.sync_copy(acc_val.at[pl.ds(row0 + off, sz)],
                      tail_rows.at[pl.ds(0, sz)])
      pltpu.sync_copy(tail_rows.at[pl.ds(0, sz)],
                      out_val.at[pl.ds(out_base + off, sz)])
      off += sz
    @pl.when(c == 0)
    def _():
      o = 0
      for sz in wb_sizes:
        pltpu.sync_copy(acc_cnt.at[pl.ds(row0 + o, sz)],
                        cnt_rows.at[pl.ds(0, sz)])
        pltpu.sync_copy(cnt_rows.at[pl.ds(0, sz)],
                        out_cnt.at[pl.ds(row0 + o, sz)])
        o += sz

  return agg(ego_halves, head, tail, etype, relw_halves)


def _combine_body(v_ref, c_ref, o_ref):
  v = v_ref[...]
  deg = jnp.maximum(c_ref[0, :, :1], 1.0)
  o_ref[...] = jnp.concatenate([v[0], v[1]], axis=1) / deg


def kernel(ego_embed, edge_index, edge_type, relation_weight):
  n_nodes, d = ego_embed.shape
  n_rel = relation_weight.shape[0]
  e = edge_index.shape[1]
  dh = d // NC
  head = edge_index[0].astype(jnp.int32)
  tail = edge_index[1].astype(jnp.int32)
  etype = edge_type.astype(jnp.int32)

  ego_halves = jnp.concatenate([ego_embed[:, :dh], ego_embed[:, dh:]], 0)
  relw_halves = jnp.concatenate(
      [relation_weight[:, :dh], relation_weight[:, dh:]], 0)

  quantum = NS * CH
  e_pad = ((e + quantum - 1) // quantum) * quantum
  if e_pad > e:
    pad = e_pad - e
    head = jnp.concatenate([head, jnp.full((pad,), n_nodes, jnp.int32)])
    tail = jnp.concatenate([tail, jnp.zeros((pad,), jnp.int32)])
    etype = jnp.concatenate([etype, jnp.zeros((pad,), jnp.int32)])
  chunks_per_tile = e_pad // quantum

  n_acc = ((n_nodes + 1 + NS * 8 - 1) // (NS * 8)) * (NS * 8)

  val_parts, cnt_part = _sc_aggregate(
      ego_halves, head, tail, etype, relw_halves,
      n_nodes, n_rel, n_acc, chunks_per_tile)
  val_parts = val_parts.reshape(NC, n_acc, dh)
  cnt_part = cnt_part.reshape(1, n_acc, L)

  br = 1000
  assert n_nodes % br == 0
  return pl.pallas_call(
      _combine_body,
      grid=(n_nodes // br,),
      in_specs=[
          pl.BlockSpec((NC, br, dh), lambda i: (0, i, 0)),
          pl.BlockSpec((1, br, L), lambda i: (0, i, 0)),
      ],
      out_specs=pl.BlockSpec((br, d), lambda i: (i, 0)),
      out_shape=jax.ShapeDtypeStruct((n_nodes, d), jnp.float32),
  )(val_parts, cnt_part)


# speedup vs baseline: 5.0134x; 1.1641x over previous
import functools

import jax
import jax.numpy as jnp
from jax import lax
from jax.experimental import pallas as pl
from jax.experimental.pallas import tpu as pltpu
from jax.experimental.pallas import tpu_sc as plsc

NC = 2
NS = 16
L = 16
CH = 128


def _sc_aggregate(ego_halves, head, tail, etype, relw_halves,
                  n_nodes, n_rel, n_acc, chunks_per_tile):
  dh = ego_halves.shape[1]
  rpt = n_acc // NS
  wb_sizes = [CH] * (rpt // CH) + ([rpt % CH] if rpt % CH else [])
  mesh = plsc.VectorSubcoreMesh(core_axis_name="c", subcore_axis_name="s",
                                num_cores=NC, num_subcores=NS)

  @functools.partial(
      pl.kernel,
      out_type=(
          jax.ShapeDtypeStruct((NC * n_acc, dh), jnp.float32),
          jax.ShapeDtypeStruct((n_acc, L), jnp.float32),
      ),
      mesh=mesh,
      compiler_params=pltpu.CompilerParams(use_tc_tiling_on_sc=False),
      scratch_types=[
          pltpu.VMEM((CH,), jnp.int32),
          pltpu.VMEM((CH,), jnp.int32),
          pltpu.VMEM((CH,), jnp.int32),
          pltpu.VMEM((CH,), jnp.int32),
          pltpu.VMEM((CH,), jnp.int32),
          pltpu.VMEM((CH,), jnp.int32),
          pltpu.VMEM((CH, dh), jnp.float32),
          pltpu.VMEM((CH, dh), jnp.float32),
          pltpu.VMEM((CH, dh), jnp.float32),
          pltpu.VMEM((CH, dh), jnp.float32),
          pltpu.VMEM((CH, L), jnp.float32),
          pltpu.VMEM_SHARED((n_rel, dh), jnp.float32),
          pltpu.VMEM_SHARED((n_acc, dh), jnp.float32),
          pltpu.VMEM_SHARED((n_acc, L), jnp.float32),
          pltpu.SemaphoreType.DMA,
          pltpu.SemaphoreType.DMA,
          pltpu.SemaphoreType.DMA,
          pltpu.SemaphoreType.DMA,
          pltpu.SemaphoreType.DMA,
          pltpu.SemaphoreType.DMA,
      ],
  )
  def agg(ego_hbm, head_hbm, tail_hbm, etype_hbm, relw_hbm,
          out_val, out_cnt, tail_idx0, tail_idx1, head_idx0, head_idx1,
          type_idx0, type_idx1, tail_rows0, tail_rows1,
          rel_rows0, rel_rows1, cnt_rows, rel_tab, acc_val, acc_cnt,
          gsem0, gsem1, rsem0, rsem1, isem0, isem1):
    tail_idx = (tail_idx0, tail_idx1)
    head_idx = (head_idx0, head_idx1)
    type_idx = (type_idx0, type_idx1)
    tail_rows = (tail_rows0, tail_rows1)
    rel_rows = (rel_rows0, rel_rows1)
    gsem = (gsem0, gsem1)
    rsem = (rsem0, rsem1)
    isem = (isem0, isem1)
    c = lax.axis_index("c")
    s = lax.axis_index("s")
    zv = jnp.zeros((L,), jnp.float32)

    def zero_body(i, _):
      for j in range(dh // L):
        tail_rows0[i, pl.ds(j * L, L)] = zv
      cnt_rows[i, :] = zv
      return 0
    lax.fori_loop(0, CH, zero_body, 0)
    row0 = s * rpt
    off = 0
    for sz in wb_sizes:
      pltpu.sync_copy(tail_rows0.at[pl.ds(0, sz)],
                      acc_val.at[pl.ds(row0 + off, sz)])
      pltpu.sync_copy(cnt_rows.at[pl.ds(0, sz)],
                      acc_cnt.at[pl.ds(row0 + off, sz)])
      off += sz

    one_hot = jnp.where(lax.iota(jnp.int32, L) == 0,
                        jnp.float32(1.0), jnp.float32(0.0))
    def oh_body(i, _):
      cnt_rows[i, :] = one_hot
      return 0
    lax.fori_loop(0, CH, oh_body, 0)

    @pl.when(s == 0)
    def _():
      pltpu.sync_copy(relw_hbm.at[pl.ds(c * n_rel, n_rel)], rel_tab)

    plsc.subcore_barrier()

    e_base = s * (chunks_per_tile * CH)
    tail_off = (c * n_nodes).astype(jnp.int32)
    n_chunks = chunks_per_tile

    def stage_idx(g, b):
      base = e_base + g * CH
      pltpu.sync_copy(tail_hbm.at[pl.ds(base, CH)], tail_idx[b])
      pltpu.sync_copy(head_hbm.at[pl.ds(base, CH)], head_idx[b])
      pltpu.sync_copy(etype_hbm.at[pl.ds(base, CH)], type_idx[b])

    def adjust_idx(b):
      def adj_body(k, _):
        sl = pl.ds(k * L, L)
        tail_idx[b][sl] = tail_idx[b][sl] + tail_off
        return 0
      lax.fori_loop(0, CH // L, adj_body, 0)

    def fire_ego(b):
      return pltpu.async_copy(ego_hbm.at[tail_idx[b]], tail_rows[b],
                              gsem[b])

    def rel_gather(b):
      pltpu.async_copy(rel_tab.at[type_idx[b]], rel_rows[b],
                       rsem[b]).wait()

    def process(b):
      def mul_body(i, _):
        for j in range(dh // L):
          sl = pl.ds(j * L, L)
          rel_rows[b][i, sl] = rel_rows[b][i, sl] * tail_rows[b][i, sl]
        return 0
      lax.fori_loop(0, CH, mul_body, 0)
      pltpu.sync_copy(rel_rows[b], acc_val.at[head_idx[b]], add=True)
      @pl.when(c == 0)
      def _():
        pltpu.sync_copy(cnt_rows, acc_cnt.at[head_idx[b]], add=True)

    def round_body(r, _):
      stage_idx(2 * r, 0)
      adjust_idx(0)
      g0 = fire_ego(0)
      stage_idx(2 * r + 1, 1)
      adjust_idx(1)
      g1 = fire_ego(1)
      g0.wait()
      rel_gather(0)
      process(0)
      g1.wait()
      rel_gather(1)
      process(1)
      return 0
    lax.fori_loop(0, n_chunks // 2, round_body, 0)

    plsc.subcore_barrier()

    out_base = c * n_acc + row0
    off = 0
    for sz in wb_sizes:
      pltpu.sync_copy(acc_val.at[pl.ds(row0 + off, sz)],
                      tail_rows0.at[pl.ds(0, sz)])
      pltpu.sync_copy(tail_rows0.at[pl.ds(0, sz)],
                      out_val.at[pl.ds(out_base + off, sz)])
      off += sz
    @pl.when(c == 0)
    def _():
      o = 0
      for sz in wb_sizes:
        pltpu.sync_copy(acc_cnt.at[pl.ds(row0 + o, sz)],
                        cnt_rows.at[pl.ds(0, sz)])
        pltpu.sync_copy(cnt_rows.at[pl.ds(0, sz)],
                        out_cnt.at[pl.ds(row0 + o, sz)])
        o += sz

  return agg(ego_halves, head, tail, etype, relw_halves)


def _combine_body(v_ref, c_ref, o_ref):
  v = v_ref[...]
  deg = jnp.maximum(c_ref[0, :, :1], 1.0)
  o_ref[...] = jnp.concatenate([v[0], v[1]], axis=1) / deg


def kernel(ego_embed, edge_index, edge_type, relation_weight):
  n_nodes, d = ego_embed.shape
  n_rel = relation_weight.shape[0]
  e = edge_index.shape[1]
  dh = d // NC
  head = edge_index[0].astype(jnp.int32)
  tail = edge_index[1].astype(jnp.int32)
  etype = edge_type.astype(jnp.int32)

  ego_halves = jnp.concatenate([ego_embed[:, :dh], ego_embed[:, dh:]], 0)
  relw_halves = jnp.concatenate(
      [relation_weight[:, :dh], relation_weight[:, dh:]], 0)

  quantum = NS * CH * 2
  e_pad = ((e + quantum - 1) // quantum) * quantum
  if e_pad > e:
    pad = e_pad - e
    head = jnp.concatenate([head, jnp.full((pad,), n_nodes, jnp.int32)])
    tail = jnp.concatenate([tail, jnp.zeros((pad,), jnp.int32)])
    etype = jnp.concatenate([etype, jnp.zeros((pad,), jnp.int32)])
  chunks_per_tile = e_pad // (NS * CH)

  n_acc = ((n_nodes + 1 + NS * 8 - 1) // (NS * 8)) * (NS * 8)

  val_parts, cnt_part = _sc_aggregate(
      ego_halves, head, tail, etype, relw_halves,
      n_nodes, n_rel, n_acc, chunks_per_tile)
  val_parts = val_parts.reshape(NC, n_acc, dh)
  cnt_part = cnt_part.reshape(1, n_acc, L)

  br = 1000
  assert n_nodes % br == 0
  return pl.pallas_call(
      _combine_body,
      grid=(n_nodes // br,),
      in_specs=[
          pl.BlockSpec((NC, br, dh), lambda i: (0, i, 0)),
          pl.BlockSpec((1, br, L), lambda i: (0, i, 0)),
      ],
      out_specs=pl.BlockSpec((br, d), lambda i: (i, 0)),
      out_shape=jax.ShapeDtypeStruct((n_nodes, d), jnp.float32),
  )(val_parts, cnt_part)


# speedup vs baseline: 7.5589x; 1.5077x over previous
import functools

import jax
import jax.numpy as jnp
from jax import lax
from jax.experimental import pallas as pl
from jax.experimental.pallas import tpu as pltpu
from jax.experimental.pallas import tpu_sc as plsc

NC = 2
NS = 16
L = 16
CH = 128


def _sc_aggregate(ego_halves, head, tail, etype, relw_halves,
                  n_nodes, n_rel, n_acc, chunks_per_tile):
  dh = ego_halves.shape[1]
  rpt = n_acc // NS
  wb_sizes = [CH] * (rpt // CH) + ([rpt % CH] if rpt % CH else [])
  mesh = plsc.VectorSubcoreMesh(core_axis_name="c", subcore_axis_name="s",
                                num_cores=NC, num_subcores=NS)

  @functools.partial(
      pl.kernel,
      out_type=(
          jax.ShapeDtypeStruct((NC * n_acc, dh), jnp.float32),
          jax.ShapeDtypeStruct((n_acc, L), jnp.float32),
      ),
      mesh=mesh,
      compiler_params=pltpu.CompilerParams(use_tc_tiling_on_sc=False),
      scratch_types=[
          pltpu.VMEM((CH,), jnp.int32),
          pltpu.VMEM((CH,), jnp.int32),
          pltpu.VMEM((CH,), jnp.int32),
          pltpu.VMEM((CH,), jnp.int32),
          pltpu.VMEM((CH,), jnp.int32),
          pltpu.VMEM((CH,), jnp.int32),
          pltpu.VMEM((CH, dh), jnp.float32),
          pltpu.VMEM((CH, dh), jnp.float32),
          pltpu.VMEM((CH, dh), jnp.float32),
          pltpu.VMEM((CH, dh), jnp.float32),
          pltpu.VMEM((CH, L), jnp.float32),
          pltpu.VMEM_SHARED((n_rel, dh), jnp.float32),
          pltpu.VMEM_SHARED((n_acc, dh), jnp.float32),
          pltpu.VMEM_SHARED((n_acc, L), jnp.float32),
          pltpu.SemaphoreType.DMA,
          pltpu.SemaphoreType.DMA,
          pltpu.SemaphoreType.DMA,
          pltpu.SemaphoreType.DMA,
          pltpu.SemaphoreType.DMA,
          pltpu.SemaphoreType.DMA,
          pltpu.SemaphoreType.DMA,
          pltpu.SemaphoreType.DMA,
      ],
  )
  def agg(ego_hbm, head_hbm, tail_hbm, etype_hbm, relw_hbm,
          out_val, out_cnt, tail_idx0, tail_idx1, head_idx0, head_idx1,
          type_idx0, type_idx1, tail_rows0, tail_rows1,
          rel_rows0, rel_rows1, cnt_rows, rel_tab, acc_val, acc_cnt,
          gsem0, gsem1, rsem0, rsem1, isem0, isem1, ssem0, ssem1):
    tail_idx = (tail_idx0, tail_idx1)
    head_idx = (head_idx0, head_idx1)
    type_idx = (type_idx0, type_idx1)
    tail_rows = (tail_rows0, tail_rows1)
    rel_rows = (rel_rows0, rel_rows1)
    gsem = (gsem0, gsem1)
    rsem = (rsem0, rsem1)
    isem = (isem0, isem1)
    ssem = (ssem0, ssem1)
    c = lax.axis_index("c")
    s = lax.axis_index("s")
    zv = jnp.zeros((L,), jnp.float32)

    def zero_body(i, _):
      for j in range(dh // L):
        tail_rows0[i, pl.ds(j * L, L)] = zv
      cnt_rows[i, :] = zv
      return 0
    lax.fori_loop(0, CH, zero_body, 0)
    row0 = s * rpt
    off = 0
    for sz in wb_sizes:
      pltpu.sync_copy(tail_rows0.at[pl.ds(0, sz)],
                      acc_val.at[pl.ds(row0 + off, sz)])
      pltpu.sync_copy(cnt_rows.at[pl.ds(0, sz)],
                      acc_cnt.at[pl.ds(row0 + off, sz)])
      off += sz

    one_hot = jnp.where(lax.iota(jnp.int32, L) == 0,
                        jnp.float32(1.0), jnp.float32(0.0))
    def oh_body(i, _):
      cnt_rows[i, :] = one_hot
      return 0
    lax.fori_loop(0, CH, oh_body, 0)

    @pl.when(s == 0)
    def _():
      pltpu.sync_copy(relw_hbm.at[pl.ds(c * n_rel, n_rel)], rel_tab)

    plsc.subcore_barrier()

    e_base = s * (chunks_per_tile * CH)
    tail_off = (c * n_nodes).astype(jnp.int32)
    n_chunks = chunks_per_tile

    def stage_idx(g, b):
      base = e_base + g * CH
      c1 = pltpu.async_copy(tail_hbm.at[pl.ds(base, CH)], tail_idx[b],
                            isem[b])
      c2 = pltpu.async_copy(head_hbm.at[pl.ds(base, CH)], head_idx[b],
                            isem[b])
      c3 = pltpu.async_copy(etype_hbm.at[pl.ds(base, CH)], type_idx[b],
                            isem[b])
      c1.wait()
      c2.wait()
      c3.wait()

    def adjust_idx(b):
      def adj_body(k, _):
        sl = pl.ds(k * L, L)
        tail_idx[b][sl] = tail_idx[b][sl] + tail_off
        return 0
      lax.fori_loop(0, CH // L, adj_body, 0)

    def fire_ego(b):
      return pltpu.async_copy(ego_hbm.at[tail_idx[b]], tail_rows[b],
                              gsem[b])

    def fire_rel(b):
      return pltpu.async_copy(rel_tab.at[type_idx[b]], rel_rows[b],
                              rsem[b])

    def fire_scatter(b):
      pltpu.async_copy(rel_rows[b], acc_val.at[head_idx[b]], ssem[b],
                       add=True)
      @pl.when(c == 0)
      def _():
        pltpu.async_copy(cnt_rows, acc_cnt.at[head_idx[b]], ssem[b],
                         add=True)

    def wait_scatter(b):
      pltpu.make_async_copy(rel_rows[b], acc_val.at[head_idx[b]],
                            ssem[b]).wait()
      @pl.when(c == 0)
      def _():
        pltpu.make_async_copy(cnt_rows, acc_cnt.at[head_idx[b]],
                              ssem[b]).wait()

    def mul(b):
      def mul_body(i, _):
        for j in range(dh // L):
          sl = pl.ds(j * L, L)
          rel_rows[b][i, sl] = rel_rows[b][i, sl] * tail_rows[b][i, sl]
        return 0
      lax.fori_loop(0, CH, mul_body, 0)

    def round_body(r, _):
      @pl.when(r > 0)
      def _():
        wait_scatter(0)
      stage_idx(2 * r, 0)
      adjust_idx(0)
      g0 = fire_ego(0)
      r0 = fire_rel(0)
      @pl.when(r > 0)
      def _():
        wait_scatter(1)
      stage_idx(2 * r + 1, 1)
      adjust_idx(1)
      g1 = fire_ego(1)
      r1 = fire_rel(1)
      g0.wait()
      r0.wait()
      mul(0)
      fire_scatter(0)
      g1.wait()
      r1.wait()
      mul(1)
      fire_scatter(1)
      return 0
    lax.fori_loop(0, n_chunks // 2, round_body, 0)
    wait_scatter(0)
    wait_scatter(1)

    plsc.subcore_barrier()

    out_base = c * n_acc + row0
    off = 0
    for sz in wb_sizes:
      pltpu.sync_copy(acc_val.at[pl.ds(row0 + off, sz)],
                      tail_rows0.at[pl.ds(0, sz)])
      pltpu.sync_copy(tail_rows0.at[pl.ds(0, sz)],
                      out_val.at[pl.ds(out_base + off, sz)])
      off += sz
    @pl.when(c == 0)
    def _():
      o = 0
      for sz in wb_sizes:
        pltpu.sync_copy(acc_cnt.at[pl.ds(row0 + o, sz)],
                        cnt_rows.at[pl.ds(0, sz)])
        pltpu.sync_copy(cnt_rows.at[pl.ds(0, sz)],
                        out_cnt.at[pl.ds(row0 + o, sz)])
        o += sz

  return agg(ego_halves, head, tail, etype, relw_halves)


def _combine_body(v_ref, c_ref, o_ref):
  v = v_ref[...]
  deg = jnp.maximum(c_ref[0, :, :1], 1.0)
  o_ref[...] = jnp.concatenate([v[0], v[1]], axis=1) / deg


def kernel(ego_embed, edge_index, edge_type, relation_weight):
  n_nodes, d = ego_embed.shape
  n_rel = relation_weight.shape[0]
  e = edge_index.shape[1]
  dh = d // NC
  head = edge_index[0].astype(jnp.int32)
  tail = edge_index[1].astype(jnp.int32)
  etype = edge_type.astype(jnp.int32)

  ego_halves = jnp.concatenate([ego_embed[:, :dh], ego_embed[:, dh:]], 0)
  relw_halves = jnp.concatenate(
      [relation_weight[:, :dh], relation_weight[:, dh:]], 0)

  quantum = NS * CH * 2
  e_pad = ((e + quantum - 1) // quantum) * quantum
  if e_pad > e:
    pad = e_pad - e
    head = jnp.concatenate([head, jnp.full((pad,), n_nodes, jnp.int32)])
    tail = jnp.concatenate([tail, jnp.zeros((pad,), jnp.int32)])
    etype = jnp.concatenate([etype, jnp.zeros((pad,), jnp.int32)])
  chunks_per_tile = e_pad // (NS * CH)

  n_acc = ((n_nodes + 1 + NS * 8 - 1) // (NS * 8)) * (NS * 8)

  val_parts, cnt_part = _sc_aggregate(
      ego_halves, head, tail, etype, relw_halves,
      n_nodes, n_rel, n_acc, chunks_per_tile)
  val_parts = val_parts.reshape(NC, n_acc, dh)
  cnt_part = cnt_part.reshape(1, n_acc, L)

  br = 1000
  assert n_nodes % br == 0
  return pl.pallas_call(
      _combine_body,
      grid=(n_nodes // br,),
      in_specs=[
          pl.BlockSpec((NC, br, dh), lambda i: (0, i, 0)),
          pl.BlockSpec((1, br, L), lambda i: (0, i, 0)),
      ],
      out_specs=pl.BlockSpec((br, d), lambda i: (i, 0)),
      out_shape=jax.ShapeDtypeStruct((n_nodes, d), jnp.float32),
  )(val_parts, cnt_part)
